# baseline (speedup 1.0000x reference)
import functools

import jax
import jax.numpy as jnp
from jax import lax
from jax.experimental import pallas as pl
from jax.experimental.pallas import tpu as pltpu
from jax.experimental.pallas import tpu_sc as plsc

_L = 16


def _newton_rsqrt(v):
    i = lax.bitcast_convert_type(v, jnp.int32)
    i = 0x5F3759DF - lax.shift_right_arithmetic(i, 1)
    y = lax.bitcast_convert_type(i, jnp.float32)
    for _ in range(4):
        y = y * (1.5 - 0.5 * v * y * y)
    return y


def _lane_sum(v):
    lanes = lax.iota(jnp.int32, _L)
    for k in (1, 2, 4, 8):
        perm = jnp.bitwise_xor(lanes, k)
        v = v + v.at[perm].get(mode="promise_in_bounds")
    return v


def _renorm_rows(rows_ref, out_ref, n_rows, d, half):
    for r in range(n_rows):
        acc = jnp.zeros((_L,), jnp.float32)
        for c in range(d // _L):
            ch = rows_ref[r, pl.ds(c * _L, _L)]
            acc = acc + ch * ch
        s2 = _lane_sum(acc)
        y = _newton_rsqrt(s2)
        norm = s2 * y
        scale = jnp.where(norm > 1.0, 1.0 / (norm + 1e-7), 1.0)
        for c in range(half // _L):
            out_ref[r, pl.ds(c * _L, _L)] = rows_ref[r, pl.ds(c * _L, _L)] * scale


def _sc_table_prep(seq_s, per_table, np16, *, s, p, d, half):
    info = plsc.get_sparse_core_info()
    nc, ns = info.num_cores, info.num_subcores
    seq_rows = 8
    n_seq_tiles = s // seq_rows
    per_rows = _L
    n_per_tiles = p // per_rows
    tab_rows = per_table.shape[0]

    @functools.partial(
        pl.kernel,
        mesh=plsc.VectorSubcoreMesh(core_axis_name="c", subcore_axis_name="s"),
        out_type=(
            jax.ShapeDtypeStruct((s, half), jnp.float32),
            jax.ShapeDtypeStruct((p, half), jnp.float32),
        ),
        scratch_types=[
            pltpu.VMEM((seq_rows, d), jnp.float32),
            pltpu.VMEM((seq_rows, half), jnp.float32),
            pltpu.VMEM((per_rows, d), jnp.float32),
            pltpu.VMEM((per_rows, half), jnp.float32),
            pltpu.VMEM((_L,), jnp.int32),
            pltpu.VMEM((_L,), jnp.int32),
            pltpu.SemaphoreType.DMA,
        ],
    )
    def prep(seq_hbm, per_hbm, np_hbm, seq_out, per_out,
             srows_v, sout_v, prows_v, pout_v, np_v, idx_v, sem):
        wid = lax.axis_index("s") * nc + lax.axis_index("c")

        @pl.when(wid < n_seq_tiles)
        def _seq():
            base = wid * seq_rows
            pltpu.sync_copy(seq_hbm.at[pl.ds(base, seq_rows)], srows_v)
            _renorm_rows(srows_v, sout_v, seq_rows, d, half)
            pltpu.sync_copy(sout_v, seq_out.at[pl.ds(base, seq_rows)])

        @pl.when((wid >= n_seq_tiles) & (wid < n_seq_tiles + n_per_tiles))
        def _person():
            pbase = (wid - n_seq_tiles) * per_rows
            pltpu.sync_copy(np_hbm, np_v)
            rowids = pbase + lax.iota(jnp.int32, _L)
            idx = jnp.minimum(rowids, np_v[...] - 1)
            idx = jnp.minimum(jnp.maximum(idx, 0), tab_rows - 1)
            idx_v[...] = idx
            pltpu.async_copy(per_hbm.at[idx_v], prows_v, sem).wait()
            _renorm_rows(prows_v, pout_v, per_rows, d, half)
            pltpu.sync_copy(pout_v, per_out.at[pl.ds(pbase, per_rows)])

    return prep(seq_s, per_table, np16)


def _enc_add_kernel(seqh_ref, perh_ref, x_ref, o_ref, enc_ref):
    @pl.when(pl.program_id(0) == 0)
    def _build_enc():
        sh = seqh_ref[...]
        ph = perh_ref[...]
        s, h = sh.shape
        p = ph.shape[0]
        enc_ref[...] = jnp.concatenate(
            [
                jnp.broadcast_to(sh[:, None, :], (s, p, h)),
                jnp.broadcast_to(ph[None, :, :], (s, p, h)),
            ],
            axis=-1,
        )

    o_ref[...] = x_ref[...] + enc_ref[...]


def kernel(x, seq_encoding, person_encoding, num_people):
    b, s, p, d = x.shape
    half = d // 2
    bb = next((c for c in (4, 2) if b % c == 0), 1)

    seq_s = seq_encoding[:s]
    np16 = jnp.full((_L,), jnp.asarray(num_people, jnp.int32))
    seq_half, per_half = _sc_table_prep(
        seq_s, person_encoding, np16, s=s, p=p, d=d, half=half
    )

    return pl.pallas_call(
        _enc_add_kernel,
        grid=(b // bb,),
        in_specs=[
            pl.BlockSpec((s, half), lambda j: (0, 0)),
            pl.BlockSpec((p, half), lambda j: (0, 0)),
            pl.BlockSpec((bb, s, p, d), lambda j: (j, 0, 0, 0)),
        ],
        out_specs=pl.BlockSpec((bb, s, p, d), lambda j: (j, 0, 0, 0)),
        out_shape=jax.ShapeDtypeStruct((b, s, p, d), x.dtype),
        scratch_shapes=[pltpu.VMEM((s, p, d), x.dtype)],
    )(seq_half, per_half, x)
